# baseline (speedup 1.0000x reference)
import functools

import jax
import jax.numpy as jnp
from jax import lax
from jax.experimental import pallas as pl
from jax.experimental.pallas import tpu as pltpu
from jax.experimental.pallas import tpu_sc as plsc

EPS = 1e-8

B = 16384
DIM = 256
SLOTS = 8192
BT = 256
HALF = SLOTS // 2


def _argmax_body(q_ref, kn_ref, idx_ref):
    qn = q_ref[...]
    kn = kn_ref[...]
    scores = lax.dot_general(
        qn, kn, (((1,), (1,)), ((), ())), preferred_element_type=jnp.float32
    )
    def half_argmax(sc, lo):
        pm = jnp.max(sc, axis=1, keepdims=True)
        pidx = jnp.argmax(sc, axis=1).astype(jnp.int32)[:, None] + lo
        return pm, pidx

    pm0, pidx0 = half_argmax(scores[:, :HALF], 0)
    pm1, pidx1 = half_argmax(scores[:, HALF:], HALF)
    m0 = pm0.astype(jnp.bfloat16).astype(jnp.float32)
    take = pm1 > m0
    idx_ref[...] = jnp.where(take, pidx1, pidx0)


def _tc_argmax(qn, kn):
    grid = (B // BT,)
    return pl.pallas_call(
        _argmax_body,
        grid=grid,
        in_specs=[
            pl.BlockSpec((BT, DIM), lambda i: (i, 0)),
            pl.BlockSpec((SLOTS, DIM), lambda i: (0, 0)),
        ],
        out_specs=pl.BlockSpec((BT, 1), lambda i: (i, 0)),
        out_shape=jax.ShapeDtypeStruct((B, 1), jnp.int32),
        compiler_params=pltpu.CompilerParams(
            dimension_semantics=("parallel",)
        ),
    )(qn, kn)


_INFO = plsc.get_sparse_core_info()
_NC, _NS = _INFO.num_cores, _INFO.num_subcores
_NW = _NC * _NS
_BPW = B // _NW
_CHUNK = 128
_NCHUNK = _BPW // _CHUNK


def _sc_gather(vals, idx):
    mesh = plsc.VectorSubcoreMesh(core_axis_name="c", subcore_axis_name="s")

    @functools.partial(
        pl.kernel,
        mesh=mesh,
        out_type=jax.ShapeDtypeStruct((B, DIM), jnp.float32),
        scratch_types=[
            pltpu.VMEM((_NCHUNK, _CHUNK), jnp.int32),
            pltpu.VMEM((_CHUNK, DIM), jnp.float32),
            pltpu.SemaphoreType.DMA,
        ],
    )
    def k(vals_hbm, idx_hbm, out_hbm, idx_v, rows_v, sem):
        wid = lax.axis_index("s") * _NC + lax.axis_index("c")
        base = wid * _BPW
        for c in range(_NCHUNK):
            off = base + c * _CHUNK
            pltpu.sync_copy(idx_hbm.at[pl.ds(off, _CHUNK)], idx_v.at[c])
            pltpu.async_copy(vals_hbm.at[idx_v.at[c]], rows_v, sem).wait()
            pltpu.sync_copy(rows_v, out_hbm.at[pl.ds(off, _CHUNK)])

    return k(vals, idx)


def kernel(query, keys, vals):
    q_norm = jnp.maximum(jnp.linalg.norm(query, axis=-1, keepdims=True), EPS)
    k_norm = jnp.maximum(jnp.linalg.norm(keys, axis=-1, keepdims=True), EPS)
    qn = (query / q_norm).astype(jnp.bfloat16)
    kn = (keys / k_norm).astype(jnp.bfloat16)
    idx = _tc_argmax(qn, kn).reshape(B)
    return _sc_gather(vals, idx)


# speedup vs baseline: 1.0062x; 1.0062x over previous
import functools

import jax
import jax.numpy as jnp
from jax import lax
from jax.experimental import pallas as pl
from jax.experimental.pallas import tpu as pltpu
from jax.experimental.pallas import tpu_sc as plsc

EPS = 1e-8

B = 16384
DIM = 256
SLOTS = 8192
BT = 512
HALF = SLOTS // 2


def _argmax_body(q_ref, kn_ref, idx_ref):
    qn = q_ref[...]
    kn = kn_ref[...]
    scores = lax.dot_general(
        qn, kn, (((1,), (1,)), ((), ())), preferred_element_type=jnp.float32
    )
    def half_argmax(sc, lo):
        pm = jnp.max(sc, axis=1, keepdims=True)
        pidx = jnp.argmax(sc, axis=1).astype(jnp.int32)[:, None] + lo
        return pm, pidx

    pm0, pidx0 = half_argmax(scores[:, :HALF], 0)
    pm1, pidx1 = half_argmax(scores[:, HALF:], HALF)
    m0 = pm0.astype(jnp.bfloat16).astype(jnp.float32)
    take = pm1 > m0
    idx_ref[...] = jnp.where(take, pidx1, pidx0)


def _tc_argmax(qn, kn):
    grid = (B // BT,)
    return pl.pallas_call(
        _argmax_body,
        grid=grid,
        in_specs=[
            pl.BlockSpec((BT, DIM), lambda i: (i, 0)),
            pl.BlockSpec((SLOTS, DIM), lambda i: (0, 0)),
        ],
        out_specs=pl.BlockSpec((BT, 1), lambda i: (i, 0)),
        out_shape=jax.ShapeDtypeStruct((B, 1), jnp.int32),
        compiler_params=pltpu.CompilerParams(
            dimension_semantics=("parallel",)
        ),
    )(qn, kn)


_INFO = plsc.get_sparse_core_info()
_NC, _NS = _INFO.num_cores, _INFO.num_subcores
_NW = _NC * _NS
_BPW = B // _NW
_CHUNK = 128
_NCHUNK = _BPW // _CHUNK


def _sc_gather(vals, idx):
    mesh = plsc.VectorSubcoreMesh(core_axis_name="c", subcore_axis_name="s")

    @functools.partial(
        pl.kernel,
        mesh=mesh,
        out_type=jax.ShapeDtypeStruct((B, DIM), jnp.float32),
        scratch_types=[
            pltpu.VMEM((_NCHUNK, _CHUNK), jnp.int32),
            pltpu.VMEM((_CHUNK, DIM), jnp.float32),
            pltpu.SemaphoreType.DMA,
        ],
    )
    def k(vals_hbm, idx_hbm, out_hbm, idx_v, rows_v, sem):
        wid = lax.axis_index("s") * _NC + lax.axis_index("c")
        base = wid * _BPW
        for c in range(_NCHUNK):
            off = base + c * _CHUNK
            pltpu.sync_copy(idx_hbm.at[pl.ds(off, _CHUNK)], idx_v.at[c])
            pltpu.async_copy(vals_hbm.at[idx_v.at[c]], rows_v, sem).wait()
            pltpu.sync_copy(rows_v, out_hbm.at[pl.ds(off, _CHUNK)])

    return k(vals, idx)


def kernel(query, keys, vals):
    q_norm = jnp.maximum(jnp.linalg.norm(query, axis=-1, keepdims=True), EPS)
    k_norm = jnp.maximum(jnp.linalg.norm(keys, axis=-1, keepdims=True), EPS)
    qn = (query / q_norm).astype(jnp.bfloat16)
    kn = (keys / k_norm).astype(jnp.bfloat16)
    idx = _tc_argmax(qn, kn).reshape(B)
    return _sc_gather(vals, idx)
